# Initial kernel scaffold; baseline (speedup 1.0000x reference)
import functools

import jax
import jax.numpy as jnp
from jax import lax
from jax.experimental import pallas as pl
from jax.experimental.pallas import tpu as pltpu
from jax.experimental.pallas import tpu_sc as plsc

N = 10000
E = 320000
FEAT = 128
HID = 16
KITER = 10
ALPHA = 0.1

NC = 2
NS = 16
NW = NC * NS
EPT = E // NW
CHUNK = 80
NCHUNK = EPT // CHUNK
RPT = N // NS
ZCH = 125
NZ = RPT // ZCH

_MESH = plsc.VectorSubcoreMesh(core_axis_name="c", subcore_axis_name="s")


def _fill_zeros(ref, nrows, width):
    z16 = jnp.zeros((16,), jnp.float32)

    def body(i, carry):
        for k in range(width // 16):
            ref[i, pl.ds(k * 16, 16)] = z16
        return carry

    lax.fori_loop(0, nrows, body, 0)


@functools.partial(
    pl.kernel,
    out_type=jax.ShapeDtypeStruct((NC, N, 16), jnp.float32),
    mesh=_MESH,
    scratch_types=[
        pltpu.VMEM((NCHUNK, CHUNK), jnp.int32),
        pltpu.VMEM((CHUNK, 16), jnp.float32),
        pltpu.VMEM((ZCH, 16), jnp.float32),
        pltpu.VMEM_SHARED((N, 16), jnp.float32),
    ],
)
def _deg_kernel(dst_hbm, cnt_hbm, idx_v, val_v, zero_v, zsh):
    c = lax.axis_index("c")
    s = lax.axis_index("s")
    pltpu.sync_copy(dst_hbm.at[c, s], idx_v)

    ones16 = jnp.ones((16,), jnp.float32)

    def fill(i, carry):
        val_v[i, pl.ds(0, 16)] = ones16
        return carry

    lax.fori_loop(0, CHUNK, fill, 0)
    _fill_zeros(zero_v, ZCH, 16)
    for r in range(NZ):
        pltpu.sync_copy(zero_v, zsh.at[pl.ds(s * RPT + r * ZCH, ZCH)])
    plsc.subcore_barrier()

    def chunk(j, carry):
        pltpu.sync_copy(val_v, zsh.at[idx_v.at[j]], add=True)
        return carry

    lax.fori_loop(0, NCHUNK, chunk, 0)
    plsc.subcore_barrier()
    pltpu.sync_copy(zsh.at[pl.ds(s * RPT, RPT)],
                    cnt_hbm.at[c, pl.ds(s * RPT, RPT)])


@functools.partial(
    pl.kernel,
    out_type=jax.ShapeDtypeStruct((NC, N, FEAT), jnp.float32),
    mesh=_MESH,
    scratch_types=[
        pltpu.VMEM((NCHUNK, CHUNK), jnp.int32),
        pltpu.VMEM((NCHUNK, CHUNK), jnp.int32),
        pltpu.VMEM((CHUNK, FEAT), jnp.float32),
        pltpu.VMEM((ZCH, FEAT), jnp.float32),
        pltpu.VMEM_SHARED((N, FEAT), jnp.float32),
        pltpu.SemaphoreType.DMA,
    ],
)
def _edge_kernel(y_hbm, src_hbm, dst_hbm, zp_hbm,
                 src_v, dst_v, rows_v, zero_v, zsh, sem):
    c = lax.axis_index("c")
    s = lax.axis_index("s")
    pltpu.sync_copy(src_hbm.at[c, s], src_v)
    pltpu.sync_copy(dst_hbm.at[c, s], dst_v)
    _fill_zeros(zero_v, ZCH, FEAT)
    for r in range(NZ):
        pltpu.sync_copy(zero_v, zsh.at[pl.ds(s * RPT + r * ZCH, ZCH)])
    plsc.subcore_barrier()

    def chunk(j, carry):
        pltpu.async_copy(y_hbm.at[src_v.at[j]], rows_v, sem).wait()
        pltpu.sync_copy(rows_v, zsh.at[dst_v.at[j]], add=True)
        return carry

    lax.fori_loop(0, NCHUNK, chunk, 0)
    plsc.subcore_barrier()
    pltpu.sync_copy(zsh.at[pl.ds(s * RPT, RPT)],
                    zp_hbm.at[c, pl.ds(s * RPT, RPT)])


_RB = 1000


def _mlp_body(x_ref, w1_ref, b1_ref, w2_ref, b2_ref, cnt_ref,
              h_ref, y0_ref, dinv_ref):
    h1 = jnp.maximum(x_ref[...] @ w1_ref[...] + b1_ref[...], 0.0)
    h = h1 @ w2_ref[...] + b2_ref[...]
    cnt = cnt_ref[...]
    deg = cnt[0, :, 0:1] + cnt[1, :, 0:1] + 1.0
    dinv = jnp.broadcast_to(lax.rsqrt(deg), h.shape)
    h_ref[...] = h
    dinv_ref[...] = dinv
    y0_ref[...] = h * dinv


def _mlp_call(x, w1t, b1, w2t, b2, cnt):
    grid = N // _RB
    return pl.pallas_call(
        _mlp_body,
        grid=(grid,),
        in_specs=[
            pl.BlockSpec((_RB, FEAT), lambda i: (i, 0)),
            pl.BlockSpec((FEAT, HID), lambda i: (0, 0)),
            pl.BlockSpec((1, HID), lambda i: (0, 0)),
            pl.BlockSpec((HID, FEAT), lambda i: (0, 0)),
            pl.BlockSpec((1, FEAT), lambda i: (0, 0)),
            pl.BlockSpec((NC, _RB, 16), lambda i: (0, i, 0)),
        ],
        out_specs=[
            pl.BlockSpec((_RB, FEAT), lambda i: (i, 0)),
            pl.BlockSpec((_RB, FEAT), lambda i: (i, 0)),
            pl.BlockSpec((_RB, FEAT), lambda i: (i, 0)),
        ],
        out_shape=[
            jax.ShapeDtypeStruct((N, FEAT), jnp.float32),
            jax.ShapeDtypeStruct((N, FEAT), jnp.float32),
            jax.ShapeDtypeStruct((N, FEAT), jnp.float32),
        ],
    )(x, w1t, b1, w2t, b2, cnt)


def _upd_body_mid(z_ref, y_ref, h_ref, dinv_ref, out_ref):
    dinv = dinv_ref[...]
    z = z_ref[...]
    t = z[0] + z[1] + y_ref[...]
    o = (1.0 - ALPHA) * dinv * t + ALPHA * h_ref[...]
    out_ref[...] = o * dinv


def _upd_body_last(z_ref, y_ref, h_ref, dinv_ref, out_ref):
    dinv = dinv_ref[...]
    z = z_ref[...]
    t = z[0] + z[1] + y_ref[...]
    o = (1.0 - ALPHA) * dinv * t + ALPHA * h_ref[...]
    m = jnp.max(o, axis=1, keepdims=True)
    lse = jnp.log(jnp.sum(jnp.exp(o - m), axis=1, keepdims=True)) + m
    out_ref[...] = o - lse


def _upd_call(zp, y, h, dinv, last):
    grid = N // _RB
    return pl.pallas_call(
        _upd_body_last if last else _upd_body_mid,
        grid=(grid,),
        in_specs=[
            pl.BlockSpec((NC, _RB, FEAT), lambda i: (0, i, 0)),
            pl.BlockSpec((_RB, FEAT), lambda i: (i, 0)),
            pl.BlockSpec((_RB, FEAT), lambda i: (i, 0)),
            pl.BlockSpec((_RB, FEAT), lambda i: (i, 0)),
        ],
        out_specs=pl.BlockSpec((_RB, FEAT), lambda i: (i, 0)),
        out_shape=jax.ShapeDtypeStruct((N, FEAT), jnp.float32),
    )(zp, y, h, dinv)


def kernel(x, edge_index, W1, b1, W2, b2):
    src = edge_index[0].reshape(NC, NS, NCHUNK, CHUNK)
    dst = edge_index[1].reshape(NC, NS, NCHUNK, CHUNK)
    cnt = _deg_kernel(dst)
    h, y0, dinv = _mlp_call(x, W1.T, b1.reshape(1, HID), W2.T,
                            b2.reshape(1, FEAT), cnt)
    y = y0
    for k in range(KITER):
        zp = _edge_kernel(y, src, dst)
        y = _upd_call(zp, y, h, dinv, last=(k == KITER - 1))
    return y


# speedup vs baseline: 20.6471x; 20.6471x over previous
import functools

import jax
import jax.numpy as jnp
from jax import lax
from jax.experimental import pallas as pl
from jax.experimental.pallas import tpu as pltpu
from jax.experimental.pallas import tpu_sc as plsc

N = 10000
NP = 10240
E = 320000
FEAT = 128
HID = 16
KITER = 10
ALPHA = 0.1

NC = 2
NS = 16
NW = NC * NS
EPT = E // NW
CHUNK = 80
NCHUNK = EPT // CHUNK
RPT = NP // NS
ZCH = 8
NZ = RPT // ZCH

_MESH = plsc.VectorSubcoreMesh(core_axis_name="c", subcore_axis_name="s")


def _fill_zeros(ref, nrows, width):
    z16 = jnp.zeros((16,), jnp.float32)

    def body(i, carry):
        for k in range(width // 16):
            ref[i, pl.ds(k * 16, 16)] = z16
        return carry

    lax.fori_loop(0, nrows, body, 0)


@functools.partial(
    pl.kernel,
    out_type=jax.ShapeDtypeStruct((NC, NP, 16), jnp.float32),
    mesh=_MESH,
    scratch_types=[
        pltpu.VMEM((NCHUNK, CHUNK), jnp.int32),
        pltpu.VMEM((CHUNK, 16), jnp.float32),
        pltpu.VMEM((ZCH, 16), jnp.float32),
        pltpu.VMEM_SHARED((NP, 16), jnp.float32),
    ],
)
def _deg_kernel(dst_hbm, cnt_hbm, idx_v, val_v, zero_v, zsh):
    c = lax.axis_index("c")
    s = lax.axis_index("s")
    pltpu.sync_copy(dst_hbm.at[c, s], idx_v)

    ones16 = jnp.ones((16,), jnp.float32)

    def fill(i, carry):
        val_v[i, pl.ds(0, 16)] = ones16
        return carry

    lax.fori_loop(0, CHUNK, fill, 0)
    _fill_zeros(zero_v, ZCH, 16)
    for r in range(NZ):
        pltpu.sync_copy(zero_v, zsh.at[pl.ds(s * RPT + r * ZCH, ZCH)])
    plsc.subcore_barrier()

    def chunk(j, carry):
        pltpu.sync_copy(val_v, zsh.at[idx_v.at[j]], add=True)
        return carry

    lax.fori_loop(0, NCHUNK, chunk, 0)
    plsc.subcore_barrier()
    pltpu.sync_copy(zsh.at[pl.ds(s * RPT, RPT)],
                    cnt_hbm.at[c, pl.ds(s * RPT, RPT)])


@functools.partial(
    pl.kernel,
    out_type=jax.ShapeDtypeStruct((NC, NP, FEAT), jnp.float32),
    mesh=_MESH,
    scratch_types=[
        pltpu.VMEM((NCHUNK, CHUNK), jnp.int32),
        pltpu.VMEM((2, CHUNK), jnp.int32),
        pltpu.VMEM((2, CHUNK), jnp.int32),
        pltpu.VMEM((CHUNK, FEAT), jnp.float32),
        pltpu.VMEM((CHUNK, FEAT), jnp.float32),
        pltpu.VMEM((ZCH, FEAT), jnp.float32),
        pltpu.SemaphoreType.DMA,
        pltpu.SemaphoreType.DMA,
        pltpu.SemaphoreType.DMA,
        pltpu.VMEM_SHARED((NP, FEAT), jnp.float32),
    ],
)
def _edge_kernel(y_hbm, pk_hbm, zp_hbm,
                 pk_v, srcu, dstu, rows0, rows1, zero_v, g0, g1, zs, zsh):
    c = lax.axis_index("c")
    s = lax.axis_index("s")
    pltpu.sync_copy(pk_hbm.at[c, s], pk_v)

    def unpack(j, slot):
        for k in range(CHUNK // 16):
            p = pk_v[j, pl.ds(k * 16, 16)]
            srcu[slot, pl.ds(k * 16, 16)] = lax.bitwise_and(p, 0xFFFF)
            dstu[slot, pl.ds(k * 16, 16)] = lax.shift_right_logical(p, 16)

    def fire(j, slot, rows, sem):
        unpack(j, slot)
        pltpu.async_copy(y_hbm.at[srcu.at[slot]], rows, sem)

    def drain_scatter(slot, rows, sem):
        pltpu.make_async_copy(y_hbm.at[srcu.at[slot]], rows, sem).wait()
        pltpu.sync_copy(rows, zsh.at[dstu.at[slot]], add=True)

    fire(0, 0, rows0, g0)
    fire(1, 1, rows1, g1)

    _fill_zeros(zero_v, ZCH, FEAT)
    zcps = [pltpu.async_copy(zero_v, zsh.at[pl.ds(s * RPT + r * ZCH, ZCH)], zs)
            for r in range(NZ)]
    for cp in zcps:
        cp.wait()
    plsc.subcore_barrier()

    def pair(jj, carry):
        j0 = 2 * jj
        drain_scatter(0, rows0, g0)
        fire(j0 + 2, 0, rows0, g0)
        drain_scatter(1, rows1, g1)
        fire(j0 + 3, 1, rows1, g1)
        return carry

    lax.fori_loop(0, (NCHUNK - 3) // 2, pair, 0)
    drain_scatter(0, rows0, g0)
    fire(NCHUNK - 1, 0, rows0, g0)
    drain_scatter(1, rows1, g1)
    drain_scatter(0, rows0, g0)

    plsc.subcore_barrier()
    pltpu.sync_copy(zsh.at[pl.ds(s * RPT, RPT)],
                    zp_hbm.at[c, pl.ds(s * RPT, RPT)])
_RB = 1000


def _mlp_body(x_ref, w1_ref, b1_ref, w2_ref, b2_ref, h_ref):
    h1 = jnp.maximum(x_ref[...] @ w1_ref[...] + b1_ref[...], 0.0)
    h_ref[...] = h1 @ w2_ref[...] + b2_ref[...]


def _mlp_call(x, w1t, b1, w2t, b2):
    grid = N // _RB
    return pl.pallas_call(
        _mlp_body,
        grid=(grid,),
        in_specs=[
            pl.BlockSpec((_RB, FEAT), lambda i: (i, 0)),
            pl.BlockSpec((FEAT, HID), lambda i: (0, 0)),
            pl.BlockSpec((1, HID), lambda i: (0, 0)),
            pl.BlockSpec((HID, FEAT), lambda i: (0, 0)),
            pl.BlockSpec((1, FEAT), lambda i: (0, 0)),
        ],
        out_specs=pl.BlockSpec((_RB, FEAT), lambda i: (i, 0)),
        out_shape=jax.ShapeDtypeStruct((N, FEAT), jnp.float32),
    )(x, w1t, b1, w2t, b2)


def _scale_body(h_ref, cnt_ref, y0_ref, a2_ref, a1_ref):
    cnt = cnt_ref[...]
    deg = cnt[0, :, 0:1] + cnt[1, :, 0:1] + 1.0
    dinv = jnp.broadcast_to(lax.rsqrt(deg), h_ref.shape)
    y0_ref[...] = h_ref[...] * dinv
    a2_ref[...] = (1.0 - ALPHA) * dinv * dinv
    a1_ref[...] = (1.0 - ALPHA) * dinv


def _scale_call(h, cnt):
    grid = N // _RB
    return pl.pallas_call(
        _scale_body,
        grid=(grid,),
        in_specs=[
            pl.BlockSpec((_RB, FEAT), lambda i: (i, 0)),
            pl.BlockSpec((NC, _RB, 16), lambda i: (0, i, 0)),
        ],
        out_specs=[
            pl.BlockSpec((_RB, FEAT), lambda i: (i, 0)),
            pl.BlockSpec((_RB, FEAT), lambda i: (i, 0)),
            pl.BlockSpec((_RB, FEAT), lambda i: (i, 0)),
        ],
        out_shape=[
            jax.ShapeDtypeStruct((N, FEAT), jnp.float32),
            jax.ShapeDtypeStruct((N, FEAT), jnp.float32),
            jax.ShapeDtypeStruct((N, FEAT), jnp.float32),
        ],
    )(h, cnt)


def _upd_body_mid(z_ref, y_ref, a2_ref, y0_ref, out_ref):
    z = z_ref[...]
    t = z[0] + z[1] + y_ref[...]
    out_ref[...] = a2_ref[...] * t + ALPHA * y0_ref[...]


def _upd_body_last(z_ref, y_ref, a1_ref, h_ref, out_ref):
    z = z_ref[...]
    t = z[0] + z[1] + y_ref[...]
    o = a1_ref[...] * t + ALPHA * h_ref[...]
    m = jnp.max(o, axis=1, keepdims=True)
    lse = jnp.log(jnp.sum(jnp.exp(o - m), axis=1, keepdims=True)) + m
    out_ref[...] = o - lse


def _upd_call(zp, y, a_arr, hy, last):
    grid = N // _RB
    return pl.pallas_call(
        _upd_body_last if last else _upd_body_mid,
        grid=(grid,),
        in_specs=[
            pl.BlockSpec((NC, _RB, FEAT), lambda i: (0, i, 0)),
            pl.BlockSpec((_RB, FEAT), lambda i: (i, 0)),
            pl.BlockSpec((_RB, FEAT), lambda i: (i, 0)),
            pl.BlockSpec((_RB, FEAT), lambda i: (i, 0)),
        ],
        out_specs=pl.BlockSpec((_RB, FEAT), lambda i: (i, 0)),
        out_shape=jax.ShapeDtypeStruct((N, FEAT), jnp.float32),
    )(zp, y, a_arr, hy)


def kernel(x, edge_index, W1, b1, W2, b2):
    src = edge_index[0]
    dst = edge_index[1]
    pk = ((dst << 16) | src).reshape(NC, NS, NCHUNK, CHUNK)
    cnt = _deg_kernel(dst.reshape(NC, NS, NCHUNK, CHUNK))
    h = _mlp_call(x, W1.T, b1.reshape(1, HID), W2.T, b2.reshape(1, FEAT))
    y0, a2, a1 = _scale_call(h, cnt)
    y = y0
    for k in range(KITER):
        zp = _edge_kernel(y, pk)
        if k < KITER - 1:
            y = _upd_call(zp, y, a2, y0, last=False)
        else:
            y = _upd_call(zp, y, a1, h, last=True)
    return y


# speedup vs baseline: 20.9243x; 1.0134x over previous
import functools

import jax
import jax.numpy as jnp
from jax import lax
from jax.experimental import pallas as pl
from jax.experimental.pallas import tpu as pltpu
from jax.experimental.pallas import tpu_sc as plsc

N = 10000
NP = 10240
E = 320000
FEAT = 128
HID = 16
KITER = 10
ALPHA = 0.1

NC = 2
NS = 16
NW = NC * NS
EPT = E // NW
CHUNK = 80
NCHUNK = EPT // CHUNK
RPT = NP // NS
ZCH = 8
NZ = RPT // ZCH

_MESH = plsc.VectorSubcoreMesh(core_axis_name="c", subcore_axis_name="s")


def _fill_zeros(ref, nrows, width):
    z16 = jnp.zeros((16,), jnp.float32)

    def body(i, carry):
        for k in range(width // 16):
            ref[i, pl.ds(k * 16, 16)] = z16
        return carry

    lax.fori_loop(0, nrows, body, 0)


@functools.partial(
    pl.kernel,
    out_type=jax.ShapeDtypeStruct((NC, NP, 16), jnp.float32),
    mesh=_MESH,
    scratch_types=[
        pltpu.VMEM((NCHUNK, CHUNK), jnp.int32),
        pltpu.VMEM((CHUNK, 16), jnp.float32),
        pltpu.VMEM((ZCH, 16), jnp.float32),
        pltpu.VMEM_SHARED((NP, 16), jnp.float32),
    ],
)
def _deg_kernel(dst_hbm, cnt_hbm, idx_v, val_v, zero_v, zsh):
    c = lax.axis_index("c")
    s = lax.axis_index("s")
    pltpu.sync_copy(dst_hbm.at[c, s], idx_v)

    ones16 = jnp.ones((16,), jnp.float32)

    def fill(i, carry):
        val_v[i, pl.ds(0, 16)] = ones16
        return carry

    lax.fori_loop(0, CHUNK, fill, 0)
    _fill_zeros(zero_v, ZCH, 16)
    for r in range(NZ):
        pltpu.sync_copy(zero_v, zsh.at[pl.ds(s * RPT + r * ZCH, ZCH)])
    plsc.subcore_barrier()

    def chunk(j, carry):
        pltpu.sync_copy(val_v, zsh.at[idx_v.at[j]], add=True)
        return carry

    lax.fori_loop(0, NCHUNK, chunk, 0)
    plsc.subcore_barrier()
    pltpu.sync_copy(zsh.at[pl.ds(s * RPT, RPT)],
                    cnt_hbm.at[c, pl.ds(s * RPT, RPT)])


@functools.partial(
    pl.kernel,
    out_type=jax.ShapeDtypeStruct((NC, NP, FEAT), jnp.float32),
    mesh=_MESH,
    scratch_types=[
        pltpu.VMEM((NCHUNK, CHUNK), jnp.int32),
        pltpu.VMEM((2, CHUNK), jnp.int32),
        pltpu.VMEM((2, CHUNK), jnp.int32),
        pltpu.VMEM((CHUNK, FEAT), jnp.float32),
        pltpu.VMEM((CHUNK, FEAT), jnp.float32),
        pltpu.VMEM((ZCH, FEAT), jnp.float32),
        pltpu.SemaphoreType.DMA,
        pltpu.SemaphoreType.DMA,
        pltpu.SemaphoreType.DMA,
        pltpu.VMEM_SHARED((NP, FEAT), jnp.float32),
    ],
)
def _edge_kernel(y_hbm, pk_hbm, zp_hbm,
                 pk_v, srcu, dstu, rows0, rows1, zero_v, g0, g1, zs, zsh):
    c = lax.axis_index("c")
    s = lax.axis_index("s")
    pltpu.sync_copy(pk_hbm.at[c, s], pk_v)

    def unpack(j, slot):
        for k in range(CHUNK // 16):
            p = pk_v[j, pl.ds(k * 16, 16)]
            srcu[slot, pl.ds(k * 16, 16)] = lax.bitwise_and(p, 0xFFFF)
            dstu[slot, pl.ds(k * 16, 16)] = lax.shift_right_logical(p, 16)

    def fire(j, slot, rows, sem):
        unpack(j, slot)
        pltpu.async_copy(y_hbm.at[srcu.at[slot]], rows, sem)

    def drain_scatter(slot, rows, sem):
        pltpu.make_async_copy(y_hbm.at[srcu.at[slot]], rows, sem).wait()
        pltpu.sync_copy(rows, zsh.at[dstu.at[slot]], add=True)

    fire(0, 0, rows0, g0)
    fire(1, 1, rows1, g1)

    _fill_zeros(zero_v, ZCH, FEAT)
    zcps = [pltpu.async_copy(zero_v, zsh.at[pl.ds(s * RPT + r * ZCH, ZCH)], zs)
            for r in range(NZ)]
    for cp in zcps:
        cp.wait()
    plsc.subcore_barrier()

    def pair(jj, carry):
        j0 = 2 * jj
        drain_scatter(0, rows0, g0)
        fire(j0 + 2, 0, rows0, g0)
        drain_scatter(1, rows1, g1)
        fire(j0 + 3, 1, rows1, g1)
        return carry

    lax.fori_loop(0, (NCHUNK - 3) // 2, pair, 0)
    drain_scatter(0, rows0, g0)
    fire(NCHUNK - 1, 0, rows0, g0)
    drain_scatter(1, rows1, g1)
    drain_scatter(0, rows0, g0)

    plsc.subcore_barrier()
    pltpu.sync_copy(zsh.at[pl.ds(s * RPT, RPT)],
                    zp_hbm.at[c, pl.ds(s * RPT, RPT)])
_RB = 2000


def _mlp_body(x_ref, w1_ref, b1_ref, w2_ref, b2_ref, h_ref):
    h1 = jnp.maximum(x_ref[...] @ w1_ref[...] + b1_ref[...], 0.0)
    h_ref[...] = h1 @ w2_ref[...] + b2_ref[...]


def _mlp_call(x, w1t, b1, w2t, b2):
    grid = N // _RB
    return pl.pallas_call(
        _mlp_body,
        grid=(grid,),
        in_specs=[
            pl.BlockSpec((_RB, FEAT), lambda i: (i, 0)),
            pl.BlockSpec((FEAT, HID), lambda i: (0, 0)),
            pl.BlockSpec((1, HID), lambda i: (0, 0)),
            pl.BlockSpec((HID, FEAT), lambda i: (0, 0)),
            pl.BlockSpec((1, FEAT), lambda i: (0, 0)),
        ],
        out_specs=pl.BlockSpec((_RB, FEAT), lambda i: (i, 0)),
        out_shape=jax.ShapeDtypeStruct((N, FEAT), jnp.float32),
    )(x, w1t, b1, w2t, b2)


def _scale_body(h_ref, cnt_ref, y0_ref, a2_ref, a1_ref):
    cnt = cnt_ref[...]
    deg = cnt[0, :, 0:1] + cnt[1, :, 0:1] + 1.0
    dinv = jnp.broadcast_to(lax.rsqrt(deg), h_ref.shape)
    y0_ref[...] = h_ref[...] * dinv
    a2_ref[...] = (1.0 - ALPHA) * dinv * dinv
    a1_ref[...] = (1.0 - ALPHA) * dinv


def _scale_call(h, cnt):
    grid = N // _RB
    return pl.pallas_call(
        _scale_body,
        grid=(grid,),
        in_specs=[
            pl.BlockSpec((_RB, FEAT), lambda i: (i, 0)),
            pl.BlockSpec((NC, _RB, 16), lambda i: (0, i, 0)),
        ],
        out_specs=[
            pl.BlockSpec((_RB, FEAT), lambda i: (i, 0)),
            pl.BlockSpec((_RB, FEAT), lambda i: (i, 0)),
            pl.BlockSpec((_RB, FEAT), lambda i: (i, 0)),
        ],
        out_shape=[
            jax.ShapeDtypeStruct((N, FEAT), jnp.float32),
            jax.ShapeDtypeStruct((N, FEAT), jnp.float32),
            jax.ShapeDtypeStruct((N, FEAT), jnp.float32),
        ],
    )(h, cnt)


def _upd_body_mid(z_ref, y_ref, a2_ref, y0_ref, out_ref):
    z = z_ref[...]
    t = z[0] + z[1] + y_ref[...]
    out_ref[...] = a2_ref[...] * t + ALPHA * y0_ref[...]


def _upd_body_last(z_ref, y_ref, a1_ref, h_ref, out_ref):
    z = z_ref[...]
    t = z[0] + z[1] + y_ref[...]
    o = a1_ref[...] * t + ALPHA * h_ref[...]
    m = jnp.max(o, axis=1, keepdims=True)
    lse = jnp.log(jnp.sum(jnp.exp(o - m), axis=1, keepdims=True)) + m
    out_ref[...] = o - lse


def _upd_call(zp, y, a_arr, hy, last):
    grid = N // _RB
    return pl.pallas_call(
        _upd_body_last if last else _upd_body_mid,
        grid=(grid,),
        in_specs=[
            pl.BlockSpec((NC, _RB, FEAT), lambda i: (0, i, 0)),
            pl.BlockSpec((_RB, FEAT), lambda i: (i, 0)),
            pl.BlockSpec((_RB, FEAT), lambda i: (i, 0)),
            pl.BlockSpec((_RB, FEAT), lambda i: (i, 0)),
        ],
        out_specs=pl.BlockSpec((_RB, FEAT), lambda i: (i, 0)),
        out_shape=jax.ShapeDtypeStruct((N, FEAT), jnp.float32),
    )(zp, y, a_arr, hy)


def kernel(x, edge_index, W1, b1, W2, b2):
    src = edge_index[0]
    dst = edge_index[1]
    pk = ((dst << 16) | src).reshape(NC, NS, NCHUNK, CHUNK)
    cnt = _deg_kernel(dst.reshape(NC, NS, NCHUNK, CHUNK))
    h = _mlp_call(x, W1.T, b1.reshape(1, HID), W2.T, b2.reshape(1, FEAT))
    y0, a2, a1 = _scale_call(h, cnt)
    y = y0
    for k in range(KITER):
        zp = _edge_kernel(y, pk)
        if k < KITER - 1:
            y = _upd_call(zp, y, a2, y0, last=False)
        else:
            y = _upd_call(zp, y, a1, h, last=True)
    return y


# speedup vs baseline: 20.9346x; 1.0005x over previous
import functools

import jax
import jax.numpy as jnp
from jax import lax
from jax.experimental import pallas as pl
from jax.experimental.pallas import tpu as pltpu
from jax.experimental.pallas import tpu_sc as plsc

N = 10000
NP = 10240
E = 320000
FEAT = 128
HID = 16
KITER = 10
ALPHA = 0.1

NC = 2
NS = 16
NW = NC * NS
EPT = E // NW
CHUNK = 80
NCHUNK = EPT // CHUNK
RPT = NP // NS
ZCH = 8
NZ = RPT // ZCH

_MESH = plsc.VectorSubcoreMesh(core_axis_name="c", subcore_axis_name="s")


def _fill_zeros(ref, nrows, width):
    z16 = jnp.zeros((16,), jnp.float32)

    def body(i, carry):
        for k in range(width // 16):
            ref[i, pl.ds(k * 16, 16)] = z16
        return carry

    lax.fori_loop(0, nrows, body, 0)


@functools.partial(
    pl.kernel,
    out_type=jax.ShapeDtypeStruct((NC, NP, 16), jnp.float32),
    mesh=_MESH,
    scratch_types=[
        pltpu.VMEM((NCHUNK, CHUNK), jnp.int32),
        pltpu.VMEM((CHUNK, 16), jnp.float32),
        pltpu.VMEM((ZCH, 16), jnp.float32),
        pltpu.VMEM_SHARED((NP, 16), jnp.float32),
    ],
)
def _deg_kernel(dst_hbm, cnt_hbm, idx_v, val_v, zero_v, zsh):
    c = lax.axis_index("c")
    s = lax.axis_index("s")
    pltpu.sync_copy(dst_hbm.at[c, s], idx_v)

    ones16 = jnp.ones((16,), jnp.float32)

    def fill(i, carry):
        val_v[i, pl.ds(0, 16)] = ones16
        return carry

    lax.fori_loop(0, CHUNK, fill, 0)
    _fill_zeros(zero_v, ZCH, 16)
    for r in range(NZ):
        pltpu.sync_copy(zero_v, zsh.at[pl.ds(s * RPT + r * ZCH, ZCH)])
    plsc.subcore_barrier()

    def chunk(j, carry):
        pltpu.sync_copy(val_v, zsh.at[idx_v.at[j]], add=True)
        return carry

    lax.fori_loop(0, NCHUNK, chunk, 0)
    plsc.subcore_barrier()
    pltpu.sync_copy(zsh.at[pl.ds(s * RPT, RPT)],
                    cnt_hbm.at[c, pl.ds(s * RPT, RPT)])


@functools.partial(
    pl.kernel,
    out_type=jax.ShapeDtypeStruct((NC, NP, FEAT), jnp.float32),
    mesh=_MESH,
    scratch_types=[
        pltpu.VMEM((NCHUNK, CHUNK), jnp.int32),
        pltpu.VMEM((2, CHUNK), jnp.int32),
        pltpu.VMEM((2, CHUNK), jnp.int32),
        pltpu.VMEM((CHUNK, FEAT), jnp.float32),
        pltpu.VMEM((CHUNK, FEAT), jnp.float32),
        pltpu.VMEM((ZCH, FEAT), jnp.float32),
        pltpu.SemaphoreType.DMA,
        pltpu.SemaphoreType.DMA,
        pltpu.SemaphoreType.DMA,
        pltpu.VMEM_SHARED((NP, FEAT), jnp.float32),
    ],
)
def _edge_kernel(y_hbm, pk_hbm, zp_hbm,
                 pk_v, srcu, dstu, rows0, rows1, zero_v, g0, g1, zs, zsh):
    c = lax.axis_index("c")
    s = lax.axis_index("s")
    pltpu.sync_copy(pk_hbm.at[c, s], pk_v)

    def unpack(j, slot):
        for k in range(CHUNK // 16):
            p = pk_v[j, pl.ds(k * 16, 16)]
            srcu[slot, pl.ds(k * 16, 16)] = lax.bitwise_and(p, 0xFFFF)
            dstu[slot, pl.ds(k * 16, 16)] = lax.shift_right_logical(p, 16)

    def fire(j, slot, rows, sem):
        unpack(j, slot)
        pltpu.async_copy(y_hbm.at[srcu.at[slot]], rows, sem)

    def drain_scatter(slot, rows, sem):
        pltpu.make_async_copy(y_hbm.at[srcu.at[slot]], rows, sem).wait()
        pltpu.sync_copy(rows, zsh.at[dstu.at[slot]], add=True)

    fire(0, 0, rows0, g0)
    fire(1, 1, rows1, g1)

    _fill_zeros(zero_v, ZCH, FEAT)
    zcps = [pltpu.async_copy(zero_v, zsh.at[pl.ds(s * RPT + r * ZCH, ZCH)], zs)
            for r in range(NZ)]
    for cp in zcps:
        cp.wait()
    plsc.subcore_barrier()

    def pair(jj, carry):
        j0 = 2 * jj
        drain_scatter(0, rows0, g0)
        fire(j0 + 2, 0, rows0, g0)
        drain_scatter(1, rows1, g1)
        fire(j0 + 3, 1, rows1, g1)
        return carry

    lax.fori_loop(0, (NCHUNK - 3) // 2, pair, 0)
    drain_scatter(0, rows0, g0)
    fire(NCHUNK - 1, 0, rows0, g0)
    drain_scatter(1, rows1, g1)
    drain_scatter(0, rows0, g0)

    plsc.subcore_barrier()
    pltpu.sync_copy(zsh.at[pl.ds(s * RPT, RPT)],
                    zp_hbm.at[c, pl.ds(s * RPT, RPT)])
_RB = 5000


def _mlp_body(x_ref, w1_ref, b1_ref, w2_ref, b2_ref, h_ref):
    h1 = jnp.maximum(x_ref[...] @ w1_ref[...] + b1_ref[...], 0.0)
    h_ref[...] = h1 @ w2_ref[...] + b2_ref[...]


def _mlp_call(x, w1t, b1, w2t, b2):
    grid = N // _RB
    return pl.pallas_call(
        _mlp_body,
        grid=(grid,),
        in_specs=[
            pl.BlockSpec((_RB, FEAT), lambda i: (i, 0)),
            pl.BlockSpec((FEAT, HID), lambda i: (0, 0)),
            pl.BlockSpec((1, HID), lambda i: (0, 0)),
            pl.BlockSpec((HID, FEAT), lambda i: (0, 0)),
            pl.BlockSpec((1, FEAT), lambda i: (0, 0)),
        ],
        out_specs=pl.BlockSpec((_RB, FEAT), lambda i: (i, 0)),
        out_shape=jax.ShapeDtypeStruct((N, FEAT), jnp.float32),
    )(x, w1t, b1, w2t, b2)


def _scale_body(h_ref, cnt_ref, y0_ref, a2_ref, a1_ref):
    cnt = cnt_ref[...]
    deg = cnt[0, :, 0:1] + cnt[1, :, 0:1] + 1.0
    dinv = jnp.broadcast_to(lax.rsqrt(deg), h_ref.shape)
    y0_ref[...] = h_ref[...] * dinv
    a2_ref[...] = (1.0 - ALPHA) * dinv * dinv
    a1_ref[...] = (1.0 - ALPHA) * dinv


def _scale_call(h, cnt):
    grid = N // _RB
    return pl.pallas_call(
        _scale_body,
        grid=(grid,),
        in_specs=[
            pl.BlockSpec((_RB, FEAT), lambda i: (i, 0)),
            pl.BlockSpec((NC, _RB, 16), lambda i: (0, i, 0)),
        ],
        out_specs=[
            pl.BlockSpec((_RB, FEAT), lambda i: (i, 0)),
            pl.BlockSpec((_RB, FEAT), lambda i: (i, 0)),
            pl.BlockSpec((_RB, FEAT), lambda i: (i, 0)),
        ],
        out_shape=[
            jax.ShapeDtypeStruct((N, FEAT), jnp.float32),
            jax.ShapeDtypeStruct((N, FEAT), jnp.float32),
            jax.ShapeDtypeStruct((N, FEAT), jnp.float32),
        ],
    )(h, cnt)


def _upd_body_mid(z_ref, y_ref, a2_ref, y0_ref, out_ref):
    z = z_ref[...]
    t = z[0] + z[1] + y_ref[...]
    out_ref[...] = a2_ref[...] * t + ALPHA * y0_ref[...]


def _upd_body_last(z_ref, y_ref, a1_ref, h_ref, out_ref):
    z = z_ref[...]
    t = z[0] + z[1] + y_ref[...]
    o = a1_ref[...] * t + ALPHA * h_ref[...]
    m = jnp.max(o, axis=1, keepdims=True)
    lse = jnp.log(jnp.sum(jnp.exp(o - m), axis=1, keepdims=True)) + m
    out_ref[...] = o - lse


def _upd_call(zp, y, a_arr, hy, last):
    grid = N // _RB
    return pl.pallas_call(
        _upd_body_last if last else _upd_body_mid,
        grid=(grid,),
        in_specs=[
            pl.BlockSpec((NC, _RB, FEAT), lambda i: (0, i, 0)),
            pl.BlockSpec((_RB, FEAT), lambda i: (i, 0)),
            pl.BlockSpec((_RB, FEAT), lambda i: (i, 0)),
            pl.BlockSpec((_RB, FEAT), lambda i: (i, 0)),
        ],
        out_specs=pl.BlockSpec((_RB, FEAT), lambda i: (i, 0)),
        out_shape=jax.ShapeDtypeStruct((N, FEAT), jnp.float32),
    )(zp, y, a_arr, hy)


def kernel(x, edge_index, W1, b1, W2, b2):
    src = edge_index[0]
    dst = edge_index[1]
    pk = ((dst << 16) | src).reshape(NC, NS, NCHUNK, CHUNK)
    cnt = _deg_kernel(dst.reshape(NC, NS, NCHUNK, CHUNK))
    h = _mlp_call(x, W1.T, b1.reshape(1, HID), W2.T, b2.reshape(1, FEAT))
    y0, a2, a1 = _scale_call(h, cnt)
    y = y0
    for k in range(KITER):
        zp = _edge_kernel(y, pk)
        if k < KITER - 1:
            y = _upd_call(zp, y, a2, y0, last=False)
        else:
            y = _upd_call(zp, y, a1, h, last=True)
    return y
